# baseline (speedup 1.0000x reference)
import functools

import jax
import jax.numpy as jnp
from jax import lax
from jax.experimental import pallas as pl
from jax.experimental.pallas import tpu as pltpu
from jax.experimental.pallas import tpu_sc as plsc

B = 1024
EMB = 16
NGRAM_CTX = 2
XDIM = NGRAM_CTX * EMB
HID = 128
VOCAB = 100000

VTILE = 1024


def _sc_gather(table, idx):
    info = plsc.get_sparse_core_info()
    nw = info.num_cores * info.num_subcores
    n = idx.shape[0]
    b_per_w = n // nw
    mesh = plsc.VectorSubcoreMesh(core_axis_name="c", subcore_axis_name="s")

    @functools.partial(
        pl.kernel,
        mesh=mesh,
        out_type=jax.ShapeDtypeStruct((n, EMB), jnp.float32),
        scratch_types=[
            pltpu.VMEM((b_per_w,), jnp.int32),
            pltpu.VMEM((b_per_w, EMB), jnp.float32),
            pltpu.SemaphoreType.DMA,
        ],
        compiler_params=pltpu.CompilerParams(use_tc_tiling_on_sc=False),
    )
    def gather_kernel(table_hbm, idx_hbm, out_hbm, idx_v, rows_v, sem):
        wid = lax.axis_index("s") * info.num_cores + lax.axis_index("c")
        base = wid * b_per_w
        pltpu.sync_copy(idx_hbm.at[pl.ds(base, b_per_w)], idx_v)
        pltpu.async_copy(table_hbm.at[idx_v], rows_v, sem).wait()
        pltpu.sync_copy(rows_v, out_hbm.at[pl.ds(base, b_per_w)])

    return gather_kernel(table, idx)


def _tc_body(x_ref, h_mat_ref, d_ref, w_ref, u_ref, b_ref, o_ref, x_scr, h_scr):
    @pl.when(pl.program_id(0) == 0)
    def _():
        x_bf = x_ref[...].astype(jnp.bfloat16)
        x_scr[...] = x_bf
        h_scr[...] = jnp.tanh(
            d_ref[...]
            + jnp.dot(
                x_bf,
                h_mat_ref[...].astype(jnp.bfloat16),
                preferred_element_type=jnp.float32,
            )
        ).astype(jnp.bfloat16)

    o_ref[...] = (
        b_ref[...]
        + jnp.dot(
            x_scr[...],
            w_ref[...].astype(jnp.bfloat16),
            preferred_element_type=jnp.float32,
        )
        + jnp.dot(
            h_scr[...],
            u_ref[...].astype(jnp.bfloat16),
            preferred_element_type=jnp.float32,
        )
    )


def _tc_forward(x, H, d, U, b, w):
    grid = (VOCAB + VTILE - 1) // VTILE
    return pl.pallas_call(
        _tc_body,
        grid=(grid,),
        in_specs=[
            pl.BlockSpec((B, XDIM), lambda j: (0, 0)),
            pl.BlockSpec((XDIM, HID), lambda j: (0, 0)),
            pl.BlockSpec((1, HID), lambda j: (0, 0)),
            pl.BlockSpec((XDIM, VTILE), lambda j: (0, j)),
            pl.BlockSpec((HID, VTILE), lambda j: (0, j)),
            pl.BlockSpec((1, VTILE), lambda j: (0, j)),
        ],
        out_specs=pl.BlockSpec((B, VTILE), lambda j: (0, j)),
        out_shape=jax.ShapeDtypeStruct((B, VOCAB), jnp.float32),
        scratch_shapes=[
            pltpu.VMEM((B, XDIM), jnp.bfloat16),
            pltpu.VMEM((B, HID), jnp.bfloat16),
        ],
        compiler_params=pltpu.CompilerParams(
            dimension_semantics=("arbitrary",),
        ),
    )(x, H, d.reshape(1, HID), w, U, b.reshape(1, VOCAB))


def kernel(input, C, H, U, d, b, w):
    idx = input.reshape(-1).astype(jnp.int32)
    x = _sc_gather(C, idx).reshape(B, XDIM)
    return _tc_forward(x, H, d, U, b, w)


# speedup vs baseline: 1.0582x; 1.0582x over previous
import functools

import jax
import jax.numpy as jnp
from jax import lax
from jax.experimental import pallas as pl
from jax.experimental.pallas import tpu as pltpu
from jax.experimental.pallas import tpu_sc as plsc

B = 1024
EMB = 16
NGRAM_CTX = 2
XDIM = NGRAM_CTX * EMB
HID = 128
VOCAB = 100000

VTILE = 4096


def _sc_gather(table, idx):
    info = plsc.get_sparse_core_info()
    nw = info.num_cores * info.num_subcores
    n = idx.shape[0]
    b_per_w = n // nw
    mesh = plsc.VectorSubcoreMesh(core_axis_name="c", subcore_axis_name="s")

    @functools.partial(
        pl.kernel,
        mesh=mesh,
        out_type=jax.ShapeDtypeStruct((n, EMB), jnp.float32),
        scratch_types=[
            pltpu.VMEM((b_per_w,), jnp.int32),
            pltpu.VMEM((b_per_w, EMB), jnp.float32),
            pltpu.SemaphoreType.DMA,
        ],
        compiler_params=pltpu.CompilerParams(use_tc_tiling_on_sc=False),
    )
    def gather_kernel(table_hbm, idx_hbm, out_hbm, idx_v, rows_v, sem):
        wid = lax.axis_index("s") * info.num_cores + lax.axis_index("c")
        base = wid * b_per_w
        pltpu.sync_copy(idx_hbm.at[pl.ds(base, b_per_w)], idx_v)
        pltpu.async_copy(table_hbm.at[idx_v], rows_v, sem).wait()
        pltpu.sync_copy(rows_v, out_hbm.at[pl.ds(base, b_per_w)])

    return gather_kernel(table, idx)


def _tc_body(x_ref, h_mat_ref, d_ref, w_ref, u_ref, b_ref, o_ref, x_scr, h_scr):
    @pl.when(pl.program_id(0) == 0)
    def _():
        x_bf = x_ref[...].astype(jnp.bfloat16)
        x_scr[...] = x_bf
        h_scr[...] = jnp.tanh(
            d_ref[...]
            + jnp.dot(
                x_bf,
                h_mat_ref[...].astype(jnp.bfloat16),
                preferred_element_type=jnp.float32,
            )
        ).astype(jnp.bfloat16)

    o_ref[...] = (
        b_ref[...]
        + jnp.dot(
            x_scr[...],
            w_ref[...].astype(jnp.bfloat16),
            preferred_element_type=jnp.float32,
        )
        + jnp.dot(
            h_scr[...],
            u_ref[...].astype(jnp.bfloat16),
            preferred_element_type=jnp.float32,
        )
    )


def _tc_forward(x, H, d, U, b, w):
    grid = (VOCAB + VTILE - 1) // VTILE
    return pl.pallas_call(
        _tc_body,
        grid=(grid,),
        in_specs=[
            pl.BlockSpec((B, XDIM), lambda j: (0, 0)),
            pl.BlockSpec((XDIM, HID), lambda j: (0, 0)),
            pl.BlockSpec((1, HID), lambda j: (0, 0)),
            pl.BlockSpec((XDIM, VTILE), lambda j: (0, j)),
            pl.BlockSpec((HID, VTILE), lambda j: (0, j)),
            pl.BlockSpec((1, VTILE), lambda j: (0, j)),
        ],
        out_specs=pl.BlockSpec((B, VTILE), lambda j: (0, j)),
        out_shape=jax.ShapeDtypeStruct((B, VOCAB), jnp.float32),
        scratch_shapes=[
            pltpu.VMEM((B, XDIM), jnp.bfloat16),
            pltpu.VMEM((B, HID), jnp.bfloat16),
        ],
        compiler_params=pltpu.CompilerParams(
            dimension_semantics=("arbitrary",),
        ),
    )(x, H, d.reshape(1, HID), w, U, b.reshape(1, VOCAB))


def kernel(input, C, H, U, d, b, w):
    idx = input.reshape(-1).astype(jnp.int32)
    x = _sc_gather(C, idx).reshape(B, XDIM)
    return _tc_forward(x, H, d, U, b, w)


# speedup vs baseline: 1.0593x; 1.0010x over previous
import functools

import jax
import jax.numpy as jnp
from jax import lax
from jax.experimental import pallas as pl
from jax.experimental.pallas import tpu as pltpu
from jax.experimental.pallas import tpu_sc as plsc

B = 1024
EMB = 16
NGRAM_CTX = 2
XDIM = NGRAM_CTX * EMB
HID = 128
VOCAB = 100000

VTILE = 4096


def _sc_gather(table, idx):
    info = plsc.get_sparse_core_info()
    nw = info.num_cores * info.num_subcores
    n = idx.shape[0]
    b_per_w = n // nw
    mesh = plsc.VectorSubcoreMesh(core_axis_name="c", subcore_axis_name="s")

    @functools.partial(
        pl.kernel,
        mesh=mesh,
        out_type=jax.ShapeDtypeStruct((n, EMB), jnp.float32),
        scratch_types=[
            pltpu.VMEM((b_per_w,), jnp.int32),
            pltpu.VMEM((b_per_w, EMB), jnp.float32),
            pltpu.SemaphoreType.DMA,
        ],
        compiler_params=pltpu.CompilerParams(use_tc_tiling_on_sc=False),
    )
    def gather_kernel(table_hbm, idx_hbm, out_hbm, idx_v, rows_v, sem):
        wid = lax.axis_index("s") * info.num_cores + lax.axis_index("c")
        base = wid * b_per_w
        pltpu.sync_copy(idx_hbm.at[pl.ds(base, b_per_w)], idx_v)
        pltpu.async_copy(table_hbm.at[idx_v], rows_v, sem).wait()
        pltpu.sync_copy(rows_v, out_hbm.at[pl.ds(base, b_per_w)])

    return gather_kernel(table, idx)


def _tc_body(x_ref, h_mat_ref, d_ref, w_ref, u_ref, b_ref, o_ref):
    x_bf = x_ref[...].astype(jnp.bfloat16)
    h_bf = jnp.tanh(
        d_ref[...]
        + jnp.dot(
            x_bf,
            h_mat_ref[...].astype(jnp.bfloat16),
            preferred_element_type=jnp.float32,
        )
    ).astype(jnp.bfloat16)
    o_ref[...] = (
        b_ref[...]
        + jnp.dot(
            x_bf,
            w_ref[...].astype(jnp.bfloat16),
            preferred_element_type=jnp.float32,
        )
        + jnp.dot(
            h_bf,
            u_ref[...].astype(jnp.bfloat16),
            preferred_element_type=jnp.float32,
        )
    )


def _tc_forward(x, H, d, U, b, w):
    grid = (VOCAB + VTILE - 1) // VTILE
    return pl.pallas_call(
        _tc_body,
        grid=(grid,),
        in_specs=[
            pl.BlockSpec((B, XDIM), lambda j: (0, 0)),
            pl.BlockSpec((XDIM, HID), lambda j: (0, 0)),
            pl.BlockSpec((1, HID), lambda j: (0, 0)),
            pl.BlockSpec((XDIM, VTILE), lambda j: (0, j)),
            pl.BlockSpec((HID, VTILE), lambda j: (0, j)),
            pl.BlockSpec((1, VTILE), lambda j: (0, j)),
        ],
        out_specs=pl.BlockSpec((B, VTILE), lambda j: (0, j)),
        out_shape=jax.ShapeDtypeStruct((B, VOCAB), jnp.float32),
        compiler_params=pltpu.CompilerParams(
            dimension_semantics=("parallel",),
        ),
    )(x, H, d.reshape(1, HID), w, U, b.reshape(1, VOCAB))


def kernel(input, C, H, U, d, b, w):
    idx = input.reshape(-1).astype(jnp.int32)
    x = _sc_gather(C, idx).reshape(B, XDIM)
    return _tc_forward(x, H, d, U, b, w)
